# Initial kernel scaffold; baseline (speedup 1.0000x reference)
import jax
import jax.numpy as jnp
from jax.experimental import pallas as pl


def kernel(x, targets, next_token_table):
    raise NotImplementedError("write your pallas kernel here")


# speedup vs baseline: 1.4301x; 1.4301x over previous
import functools

import jax
import jax.numpy as jnp
from jax import lax
from jax.experimental import pallas as pl
from jax.experimental.pallas import tpu as pltpu
from jax.experimental.pallas import tpu_sc as plsc

VOCAB = 1000
N_ROWS = 51200

NC, NS, L = 2, 16, 16
NW = NC * NS
ROWS_PER_W = N_ROWS // NW
CHUNK = 32
NCHUNK = ROWS_PER_W // CHUNK
NBUF = 2


def _lse_body(table_ref, lse_ref):
    t = table_ref[...]
    m = jnp.max(t, axis=1, keepdims=True)
    s = jnp.sum(jnp.exp(t - m), axis=1, keepdims=True)
    lse_ref[...] = jnp.log(s) + m


def _lse_table(table):
    return pl.pallas_call(
        _lse_body,
        out_shape=jax.ShapeDtypeStruct((VOCAB, 1), jnp.float32),
    )(table)


def _loss_body(part_ref, out_ref):
    out_ref[...] = jnp.sum(part_ref[...], keepdims=True) * (1.0 / N_ROWS)


def _loss_reduce(partials):
    return pl.pallas_call(
        _loss_body,
        out_shape=jax.ShapeDtypeStruct((1, 1), jnp.float32),
    )(partials)


def _sc_body(table_hbm, x_hbm, t_hbm, lse_hbm, out_hbm, part_hbm,
             idx_v, tgt_v, lse_v, acc_v, rows0, rows1, sem0, sem1):
    wid = lax.axis_index("s") * NC + lax.axis_index("c")
    base = wid * ROWS_PER_W

    pltpu.sync_copy(x_hbm.at[pl.ds(base, ROWS_PER_W)], idx_v)
    pltpu.sync_copy(t_hbm.at[pl.ds(base, ROWS_PER_W)], tgt_v)
    pltpu.sync_copy(lse_hbm, lse_v)
    acc_v[...] = jnp.zeros((L,), jnp.float32)

    rows = (rows0, rows1)
    sems = (sem0, sem1)

    def _gather_start(i, b):
        pltpu.async_copy(
            table_hbm.at[idx_v.at[pl.ds(i * CHUNK, CHUNK)]], rows[b], sems[b])

    def _gather_wait(i, b):
        pltpu.make_async_copy(
            table_hbm.at[idx_v.at[pl.ds(i * CHUNK, CHUNK)]], rows[b], sems[b]
        ).wait()

    def _consume(i, b):
        for j in range(CHUNK // L):
            xv = idx_v[pl.ds(i * CHUNK + j * L, L)]
            tv = tgt_v[pl.ds(i * CHUNK + j * L, L)]
            lsev = plsc.load_gather(lse_v, [xv])
            rowids = lax.iota(jnp.int32, L) + j * L
            picked = plsc.load_gather(rows[b], [rowids, tv])
            acc_v[...] = acc_v[...] + (lsev - picked)
        pltpu.sync_copy(rows[b], out_hbm.at[pl.ds(base + i * CHUNK, CHUNK)])

    _gather_start(0, 0)

    def _outer(g, carry):
        for b in range(NBUF):
            i = g * NBUF + b
            nxt = i + 1

            @pl.when(nxt < NCHUNK)
            def _():
                _gather_start(nxt, (b + 1) % NBUF)

            _gather_wait(i, b)
            _consume(i, b)
        return carry

    lax.fori_loop(0, NCHUNK // NBUF, _outer, 0)

    pltpu.sync_copy(acc_v, part_hbm.at[wid])


@functools.lru_cache(maxsize=1)
def _sc_gather():
    return pl.kernel(
        _sc_body,
        out_type=(
            jax.ShapeDtypeStruct((N_ROWS, VOCAB), jnp.float32),
            jax.ShapeDtypeStruct((NW, L), jnp.float32),
        ),
        mesh=plsc.VectorSubcoreMesh(
            core_axis_name="c", subcore_axis_name="s", num_cores=NC,
            num_subcores=NS),
        scratch_types=(
            pltpu.VMEM((ROWS_PER_W,), jnp.int32),
            pltpu.VMEM((ROWS_PER_W,), jnp.int32),
            pltpu.VMEM((VOCAB,), jnp.float32),
            pltpu.VMEM((L,), jnp.float32),
            pltpu.VMEM((CHUNK, VOCAB), jnp.float32),
            pltpu.VMEM((CHUNK, VOCAB), jnp.float32),
            pltpu.SemaphoreType.DMA,
            pltpu.SemaphoreType.DMA,
        ),
        compiler_params=pltpu.CompilerParams(
            needs_layout_passes=False, use_tc_tiling_on_sc=False),
    )


def kernel(x, targets, next_token_table):
    B, T = x.shape
    xf = x.reshape(N_ROWS).astype(jnp.int32)
    tf = targets.reshape(N_ROWS).astype(jnp.int32)
    lse = _lse_table(next_token_table).reshape(VOCAB)
    logits_flat, partials = _sc_gather()(next_token_table, xf, tf, lse)
    loss = _loss_reduce(partials)
    return logits_flat.reshape(B, T, VOCAB), loss[0, 0]
